# baseline (speedup 1.0000x reference)
import functools

import jax
import jax.numpy as jnp
from jax import lax
from jax.experimental import pallas as pl
from jax.experimental.pallas import tpu as pltpu
from jax.experimental.pallas import tpu_sc as plsc

N = 10000
E = 320000
IN_DIM = 128
HID = 128
CLS = 32

_PREC = lax.Precision.HIGHEST

_NW = 32
_R = 320
_NPAD = _NW * _R
_C = 8000
_G = 128
_F = IN_DIM // 16


def _pool_body(m_hbm, src_hbm, dst_hbm, out_hbm,
               dst_buf, src_buf, wl_src, wl_dstl, idx_g, rows, pooled, sem):
    i32 = jnp.int32
    wid = lax.axis_index("s") * i32(2) + lax.axis_index("c")
    lo = wid * i32(_R)
    zi = jnp.zeros((16,), jnp.int32)
    zf = jnp.zeros((16,), jnp.float32)
    ri = jnp.full((16,), _R, jnp.int32)

    def _z_pooled(r, _):
        for f in range(_F):
            pooled[r, pl.ds(f * 16, 16)] = zf
        return 0
    lax.fori_loop(i32(0), i32(_R + 1), _z_pooled, 0)

    def _z_wl(i, _):
        wl_src[pl.ds(i * i32(16), 16)] = zi
        wl_dstl[pl.ds(i * i32(16), 16)] = ri
        return 0
    lax.fori_loop(i32(0), i32((_C + 16) // 16), _z_wl, 0)

    def _scan_chunk(ch, _):
        base = ch * i32(_C)
        pltpu.sync_copy(dst_hbm.at[pl.ds(base, _C)], dst_buf)
        pltpu.sync_copy(src_hbm.at[pl.ds(base, _C)], src_buf)

        def _scan_v(v, cnt):
            d = dst_buf[pl.ds(v * i32(16), 16)]
            msk = (d >= lo) & (d < lo + i32(_R))
            n = plsc.all_reduce_population_count(msk)[0]

            def _store(c):
                s = src_buf[pl.ds(v * i32(16), 16)]
                plsc.store_compressed(wl_src.at[pl.ds(c, 16)], s, mask=msk)
                plsc.store_compressed(wl_dstl.at[pl.ds(c, 16)], d - lo, mask=msk)
                return c + n

            return lax.cond(n > 0, _store, lambda c: c, cnt)

        cnt = lax.fori_loop(i32(0), i32(_C // 16), _scan_v, i32(0))

        nb = (cnt + i32(_G - 1)) // i32(_G)

        def _batch(b, _):
            bG = b * i32(_G)
            for i in range(_G // 16):
                idx_g[pl.ds(i * 16, 16)] = wl_src[pl.ds(bG + i32(i * 16), 16)]
            cp = pltpu.async_copy(m_hbm.at[idx_g], rows, sem)
            cp.wait()

            def _edge16(q, _):
                q16 = q * i32(16)
                dls = wl_dstl[pl.ds(bG + q16, 16)]
                for l in range(16):
                    dl = dls[l]
                    r = q16 + i32(l)
                    for f in range(_F):
                        sl = pl.ds(f * 16, 16)
                        pooled[dl, sl] = jnp.maximum(pooled[dl, sl],
                                                     rows[r, sl])
                return 0
            lax.fori_loop(i32(0), i32(_G // 16), _edge16, 0)
            return 0

        lax.fori_loop(i32(0), nb, _batch, 0)
        return 0

    lax.fori_loop(i32(0), i32(E // _C), _scan_chunk, 0)
    pltpu.sync_copy(pooled.at[pl.ds(0, _R)], out_hbm.at[pl.ds(lo, _R)])


@functools.partial(jax.jit, static_argnames=())
def _segment_max_sc(m, src, dst):
    mesh = plsc.VectorSubcoreMesh(core_axis_name="c", subcore_axis_name="s")
    call = pl.kernel(
        _pool_body,
        out_type=jax.ShapeDtypeStruct((_NPAD, IN_DIM), jnp.float32),
        mesh=mesh,
        scratch_types=[
            pltpu.VMEM((_C,), jnp.int32),
            pltpu.VMEM((_C,), jnp.int32),
            pltpu.VMEM((_C + 16,), jnp.int32),
            pltpu.VMEM((_C + 16,), jnp.int32),
            pltpu.VMEM((_G,), jnp.int32),
            pltpu.VMEM((_G, IN_DIM), jnp.float32),
            pltpu.VMEM((_R + 1, IN_DIM), jnp.float32),
            pltpu.SemaphoreType.DMA,
        ],
        compiler_params=pltpu.CompilerParams(needs_layout_passes=False),
    )
    return call(m, src, dst)[:N]


def _pre_body(h_ref, wp_ref, bp_ref, ws_ref, m_ref, hs_ref):
    h = h_ref[...]
    m_ref[...] = jnp.maximum(
        jnp.dot(h, wp_ref[...].T, precision=_PREC) + bp_ref[...], 0.0)
    hs_ref[...] = jnp.dot(h, ws_ref[...].T, precision=_PREC)


def _mid_body(hs_ref, p_ref, wn_ref, b_ref, wp2_ref, bp2_ref, ws2_ref,
              m2_ref, hs2_ref):
    x = hs_ref[...] + jnp.dot(p_ref[...], wn_ref[...].T, precision=_PREC) + b_ref[...]
    h1 = jnp.where(x > 0, x, jnp.exp(jnp.minimum(x, 0.0)) - 1.0)
    m2_ref[...] = jnp.maximum(
        jnp.dot(h1, wp2_ref[...].T, precision=_PREC) + bp2_ref[...], 0.0)
    hs2_ref[...] = jnp.dot(h1, ws2_ref[...].T, precision=_PREC)


def _post_body(hs2_ref, p2_ref, wn2_ref, b2_ref, out_ref):
    logits = (hs2_ref[...] + jnp.dot(p2_ref[...], wn2_ref[...].T, precision=_PREC)
              + b2_ref[...])
    out_ref[...] = jnp.mean(logits, axis=1, keepdims=True)


def _segment_max(m, src, dst):
    return _segment_max_sc(m, src, dst)


def kernel(h, edge_index, Wp1, bp1, Ws1, Wn1, b1, Wp2, bp2, Ws2, Wn2, b2):
    src = edge_index[0].astype(jnp.int32)
    dst = edge_index[1].astype(jnp.int32)

    m1, hs1 = pl.pallas_call(
        _pre_body,
        out_shape=[jax.ShapeDtypeStruct((N, IN_DIM), jnp.float32),
                   jax.ShapeDtypeStruct((N, HID), jnp.float32)],
    )(h, Wp1, bp1, Ws1)

    pooled1 = _segment_max(m1, src, dst)

    m2, hs2 = pl.pallas_call(
        _mid_body,
        out_shape=[jax.ShapeDtypeStruct((N, HID), jnp.float32),
                   jax.ShapeDtypeStruct((N, CLS), jnp.float32)],
    )(hs1, pooled1, Wn1, b1, Wp2, bp2, Ws2)

    pooled2 = _segment_max(m2, src, dst)

    out = pl.pallas_call(
        _post_body,
        out_shape=jax.ShapeDtypeStruct((N, 1), jnp.float32),
    )(hs2, pooled2, Wn2, b2)
    return out.reshape(N)
